# Initial kernel scaffold; baseline (speedup 1.0000x reference)
import jax
import jax.numpy as jnp
from jax.experimental import pallas as pl


def kernel(codes, codebooks, scales):
    raise NotImplementedError("write your pallas kernel here")


# speedup vs baseline: 49.7192x; 49.7192x over previous
import functools

import jax
import jax.numpy as jnp
from jax import lax
from jax.experimental import pallas as pl
from jax.experimental.pallas import tpu as pltpu
from jax.experimental.pallas import tpu_sc as plsc

CODEBOOK_NUM = 2
CENTROIDS = 256
BLOCK = 64
ROWS = 4096
COLS = 4096
NUMEL = ROWS * COLS
NVEC = NUMEL // 4
NBLOCKS = NUMEL // BLOCK

NUM_WORKERS = 32
NV_W = NVEC // NUM_WORKERS
VC = 8192
NCHUNK = NV_W // VC
ITERS = VC // 16


def _body(codes_hbm, table_hbm, scales_hbm, out_hbm,
          table_v, codes_v, scales_v, out_v):
    nc = plsc.get_sparse_core_info().num_cores
    wid = lax.axis_index("s") * nc + lax.axis_index("c")
    base = wid * NV_W

    pltpu.sync_copy(table_hbm, table_v)

    cb_off = jnp.where(wid >= NUM_WORKERS // CODEBOOK_NUM,
                       CENTROIDS * 4, 0).astype(jnp.int32)
    lane = lax.iota(jnp.int32, 16)
    lane4 = lane * 4

    def chunk_body(k, _):
        cstart = pl.multiple_of(base + k * VC, VC)
        pltpu.sync_copy(codes_hbm.at[pl.ds(cstart, VC)], codes_v)
        pltpu.sync_copy(scales_hbm.at[pl.ds(pl.multiple_of(cstart // 16, VC // 16), VC // 16)],
                        scales_v)

        def vec_body(u, _):
            svec = scales_v[pl.ds(u * 16, 16)]
            for i in range(16):
                c = codes_v[pl.ds(u * 256 + i * 16, 16)]
                s = svec[i]
                gidx = c * 4 + cb_off
                sidx = lane4 + u * 1024 + i * 64
                for j in range(4):
                    vals = plsc.load_gather(table_v, [gidx + j])
                    plsc.store_scatter(out_v, [sidx + j], vals * s)
            return 0

        lax.fori_loop(0, ITERS // 16, vec_body, 0)
        pltpu.sync_copy(out_v, out_hbm.at[pl.ds(cstart * 4, VC * 4)])
        return 0

    lax.fori_loop(0, NCHUNK, chunk_body, 0)


@jax.jit
def _dequant(codes_flat, table_flat, scales_flat):
    mesh = plsc.VectorSubcoreMesh(core_axis_name="c", subcore_axis_name="s")
    run = pl.kernel(
        _body,
        out_type=jax.ShapeDtypeStruct((NUMEL,), jnp.float32),
        mesh=mesh,
        scratch_types=[
            pltpu.VMEM((CODEBOOK_NUM * CENTROIDS * 4,), jnp.float32),
            pltpu.VMEM((VC,), jnp.int32),
            pltpu.VMEM((VC // 16,), jnp.float32),
            pltpu.VMEM((VC * 4,), jnp.float32),
        ],
        compiler_params=pltpu.CompilerParams(needs_layout_passes=False),
    )
    return run(codes_flat, table_flat, scales_flat)


def kernel(codes, codebooks, scales):
    codes_flat = codes.reshape(NVEC)
    table_flat = codebooks.reshape(CODEBOOK_NUM * CENTROIDS * 4)
    scales_flat = scales.reshape(NBLOCKS)
    out = _dequant(codes_flat, table_flat, scales_flat)
    return out.reshape(ROWS, COLS)


# speedup vs baseline: 113.5570x; 2.2840x over previous
import functools

import jax
import jax.numpy as jnp
from jax import lax
from jax.experimental import pallas as pl
from jax.experimental.pallas import tpu as pltpu
from jax.experimental.pallas import tpu_sc as plsc

CODEBOOK_NUM = 2
CENTROIDS = 256
BLOCK = 64
ROWS = 4096
COLS = 4096
NUMEL = ROWS * COLS
NVEC = NUMEL // 4
NBLOCKS = NUMEL // BLOCK

NUM_WORKERS = 32
NV_W = NVEC // NUM_WORKERS
VC = 8192
NCHUNK = NV_W // VC
ITERS = VC // 16


def _vperm(vec, idx):
    dnums = lax.GatherDimensionNumbers(
        offset_dims=(), collapsed_slice_dims=(0,), start_index_map=(0,))
    return lax.gather(vec, idx[:, None], dimension_numbers=dnums,
                      slice_sizes=(1,),
                      mode=lax.GatherScatterMode.PROMISE_IN_BOUNDS)


def _body(codes_hbm, table_hbm, scales_hbm, out_hbm,
          table_v, codes_v, scales_v, out_v):
    nc = plsc.get_sparse_core_info().num_cores
    wid = lax.axis_index("s") * nc + lax.axis_index("c")
    base = wid * NV_W

    pltpu.sync_copy(table_hbm, table_v)

    cb_off = jnp.where(wid >= NUM_WORKERS // CODEBOOK_NUM,
                       CENTROIDS * 4, 0).astype(jnp.int32)
    lane = lax.iota(jnp.int32, 16)
    laneoff = (lane & 3) + cb_off
    rep = [(lane >> 2) + 4 * r for r in range(4)]

    def chunk_body(k, _):
        cstart = pl.multiple_of(base + k * VC, VC)
        pltpu.sync_copy(codes_hbm.at[pl.ds(cstart, VC)], codes_v)
        pltpu.sync_copy(scales_hbm.at[pl.ds(pl.multiple_of(cstart // 16, VC // 16), VC // 16)],
                        scales_v)

        @plsc.parallel_loop(0, ITERS, 1, unroll=8)
        def vec_body(t):
            c = codes_v[pl.ds(t * 16, 16)]
            svec = plsc.load_gather(scales_v, [jnp.full((16,), t, jnp.int32)])
            c4 = c * 4
            for r in range(4):
                crep = _vperm(c4, rep[r])
                vals = plsc.load_gather(table_v, [crep + laneoff])
                out_v[pl.ds(t * 64 + r * 16, 16)] = vals * svec
        pltpu.sync_copy(out_v, out_hbm.at[pl.ds(cstart * 4, VC * 4)])
        return 0

    lax.fori_loop(0, NCHUNK, chunk_body, 0)


@jax.jit
def _dequant(codes_flat, table_flat, scales_flat):
    mesh = plsc.VectorSubcoreMesh(core_axis_name="c", subcore_axis_name="s")
    run = pl.kernel(
        _body,
        out_type=jax.ShapeDtypeStruct((NUMEL,), jnp.float32),
        mesh=mesh,
        scratch_types=[
            pltpu.VMEM((CODEBOOK_NUM * CENTROIDS * 4,), jnp.float32),
            pltpu.VMEM((VC,), jnp.int32),
            pltpu.VMEM((VC // 16,), jnp.float32),
            pltpu.VMEM((VC * 4,), jnp.float32),
        ],
        compiler_params=pltpu.CompilerParams(needs_layout_passes=False),
    )
    return run(codes_flat, table_flat, scales_flat)


def kernel(codes, codebooks, scales):
    codes_flat = codes.reshape(NVEC)
    table_flat = codebooks.reshape(CODEBOOK_NUM * CENTROIDS * 4)
    scales_flat = scales.reshape(NBLOCKS)
    out = _dequant(codes_flat, table_flat, scales_flat)
    return out.reshape(ROWS, COLS)


# speedup vs baseline: 211.2459x; 1.8603x over previous
import jax
import jax.numpy as jnp
from jax import lax
from jax.experimental import pallas as pl
from jax.experimental.pallas import tpu as pltpu
from jax.experimental.pallas import tpu_sc as plsc

CODEBOOK_NUM = 2
CENTROIDS = 256
BLOCK = 64
ROWS = 4096
COLS = 4096
NUMEL = ROWS * COLS
NVEC = NUMEL // 4
NBLOCKS = NUMEL // BLOCK

NUM_WORKERS = 32
NBANDS = ROWS // 8
BANDS_W = NBANDS // NUM_WORKERS
VC = NUMEL // NBANDS // 4
ITERS = VC // 16


def _vperm(vec, idx):
    dnums = lax.GatherDimensionNumbers(
        offset_dims=(), collapsed_slice_dims=(0,), start_index_map=(0,))
    return lax.gather(vec, idx[:, None], dimension_numbers=dnums,
                      slice_sizes=(1,),
                      mode=lax.GatherScatterMode.PROMISE_IN_BOUNDS)


def _body(codes_hbm, table_hbm, scales_hbm, out_hbm,
          table_v, codes_v, scales_v, out_v):
    nc = plsc.get_sparse_core_info().num_cores
    wid = lax.axis_index("s") * nc + lax.axis_index("c")

    pltpu.sync_copy(table_hbm, table_v)

    cb_row = wid // (NUM_WORKERS // CODEBOOK_NUM)
    cb_off = (cb_row * CENTROIDS * 4).astype(jnp.int32)
    lane = lax.iota(jnp.int32, 16)
    laneoff = (lane & 3) + cb_off
    rep = [(lane >> 2) + 4 * r for r in range(4)]

    def band_body(k, _):
        b = wid * BANDS_W + k
        col0 = pl.multiple_of((b % (NBANDS // CODEBOOK_NUM)) * VC, VC)
        pltpu.sync_copy(codes_hbm.at[cb_row, pl.ds(col0, VC)], codes_v)
        pltpu.sync_copy(
            scales_hbm.at[pl.ds(pl.multiple_of(b * (VC // 16), VC // 16),
                                VC // 16)],
            scales_v)

        @plsc.parallel_loop(0, ITERS, 1, unroll=8)
        def vec_body(t):
            c = codes_v[pl.ds(t * 16, 16)]
            svec = plsc.load_gather(scales_v, [jnp.full((16,), t, jnp.int32)])
            c4 = c * 4
            row = t >> 6
            base = (t & 63) * 64
            for r in range(4):
                crep = _vperm(c4, rep[r])
                vals = plsc.load_gather(table_v, [crep + laneoff])
                out_v[row, pl.ds(base + r * 16, 16)] = vals * svec

        pltpu.sync_copy(out_v,
                        out_hbm.at[pl.ds(pl.multiple_of(b * 8, 8), 8), :])
        return 0

    lax.fori_loop(0, BANDS_W, band_body, 0)


@jax.jit
def _dequant(codes, table_flat, scales_flat):
    mesh = plsc.VectorSubcoreMesh(core_axis_name="c", subcore_axis_name="s")
    run = pl.kernel(
        _body,
        out_type=jax.ShapeDtypeStruct((ROWS, COLS), jnp.float32),
        mesh=mesh,
        scratch_types=[
            pltpu.VMEM((CODEBOOK_NUM * CENTROIDS * 4,), jnp.float32),
            pltpu.VMEM((VC,), jnp.int32),
            pltpu.VMEM((VC // 16,), jnp.float32),
            pltpu.VMEM((8, COLS), jnp.float32),
        ],
        compiler_params=pltpu.CompilerParams(needs_layout_passes=False,
                                             use_tc_tiling_on_sc=True),
    )
    return run(codes, table_flat, scales_flat)


def kernel(codes, codebooks, scales):
    table_flat = codebooks.reshape(CODEBOOK_NUM * CENTROIDS * 4)
    scales_flat = scales.reshape(NBLOCKS)
    return _dequant(codes, table_flat, scales_flat)


# speedup vs baseline: 368.8607x; 1.7461x over previous
import jax
import jax.numpy as jnp
from jax import lax
from jax.experimental import pallas as pl
from jax.experimental.pallas import tpu as pltpu
from jax.experimental.pallas import tpu_sc as plsc

CODEBOOK_NUM = 2
CENTROIDS = 256
BLOCK = 64
ROWS = 4096
COLS = 4096
NUMEL = ROWS * COLS
NVEC = NUMEL // 4
NBLOCKS = NUMEL // BLOCK

NUM_WORKERS = 32
NBANDS = ROWS // 8
BANDS_W = NBANDS // NUM_WORKERS
VC = NUMEL // NBANDS // 4
ITERS = VC // 16


def _vperm(vec, idx):
    dnums = lax.GatherDimensionNumbers(
        offset_dims=(), collapsed_slice_dims=(0,), start_index_map=(0,))
    return lax.gather(vec, idx[:, None], dimension_numbers=dnums,
                      slice_sizes=(1,),
                      mode=lax.GatherScatterMode.PROMISE_IN_BOUNDS)


def _body(codes_hbm, table_hbm, scales_hbm, out_hbm,
          table_v, codes_v0, codes_v1, scales_v0, scales_v1,
          out_v0, out_v1, in_sem0, in_sem1, out_sem0, out_sem1):
    nc = plsc.get_sparse_core_info().num_cores
    wid = lax.axis_index("s") * nc + lax.axis_index("c")

    pltpu.sync_copy(table_hbm, table_v)

    cb_row = wid // (NUM_WORKERS // CODEBOOK_NUM)
    cb_off = (cb_row * CENTROIDS * 4).astype(jnp.int32)
    lane = lax.iota(jnp.int32, 16)
    laneoff = (lane & 3) + cb_off
    rep = [(lane >> 2) + 4 * r for r in range(4)]

    cbufs = (codes_v0, codes_v1)
    sbufs = (scales_v0, scales_v1)
    obufs = (out_v0, out_v1)
    in_sems = (in_sem0, in_sem1)
    out_sems = (out_sem0, out_sem1)

    def start_in(k, cbuf, sbuf, sem):
        b = wid * BANDS_W + k
        col0 = pl.multiple_of((b % (NBANDS // CODEBOOK_NUM)) * VC, VC)
        pltpu.async_copy(codes_hbm.at[cb_row, pl.ds(col0, VC)], cbuf, sem)
        pltpu.async_copy(
            scales_hbm.at[pl.ds(pl.multiple_of(b * (VC // 16), VC // 16),
                                VC // 16)],
            sbuf, sem)

    def out_slice(b):
        return out_hbm.at[pl.ds(pl.multiple_of(b * 8, 8), 8), :]

    start_in(0, cbufs[0], sbufs[0], in_sems[0])
    start_in(1, cbufs[1], sbufs[1], in_sems[1])

    def step(m, _):
        for phase in range(2):
            k = 2 * m + phase
            cbuf, sbuf = cbufs[phase], sbufs[phase]
            obuf = obufs[phase]
            in_sem, out_sem = in_sems[phase], out_sems[phase]
            b = wid * BANDS_W + k

            pltpu.make_async_copy(codes_hbm.at[cb_row, pl.ds(0, VC)],
                                  cbuf, in_sem).wait()
            pltpu.make_async_copy(scales_hbm.at[pl.ds(0, VC // 16)],
                                  sbuf, in_sem).wait()

            @pl.when(m > 0)
            def _():
                pltpu.make_async_copy(obuf, out_slice(b), out_sem).wait()

            @plsc.parallel_loop(0, ITERS, 1, unroll=8)
            def vec_body(t):
                c = cbuf[pl.ds(t * 16, 16)]
                svec = plsc.load_gather(sbuf,
                                        [jnp.full((16,), t, jnp.int32)])
                c4 = c * 4
                row = t >> 6
                base = (t & 63) * 64
                for r in range(4):
                    crep = _vperm(c4, rep[r])
                    vals = plsc.load_gather(table_v, [crep + laneoff])
                    obuf[row, pl.ds(base + r * 16, 16)] = vals * svec

            pltpu.async_copy(obuf, out_slice(b), out_sem)

            @pl.when(m < BANDS_W // 2 - 1)
            def _():
                start_in(k + 2, cbuf, sbuf, in_sem)
        return 0

    lax.fori_loop(0, BANDS_W // 2, step, 0)
    for phase in range(2):
        pltpu.make_async_copy(obufs[phase], out_slice(0),
                              out_sems[phase]).wait()


@jax.jit
def _dequant(codes, table_flat, scales_flat):
    mesh = plsc.VectorSubcoreMesh(core_axis_name="c", subcore_axis_name="s")
    run = pl.kernel(
        _body,
        out_type=jax.ShapeDtypeStruct((ROWS, COLS), jnp.float32),
        mesh=mesh,
        scratch_types=[
            pltpu.VMEM((CODEBOOK_NUM * CENTROIDS * 4,), jnp.float32),
            pltpu.VMEM((VC,), jnp.int32),
            pltpu.VMEM((VC,), jnp.int32),
            pltpu.VMEM((VC // 16,), jnp.float32),
            pltpu.VMEM((VC // 16,), jnp.float32),
            pltpu.VMEM((8, COLS), jnp.float32),
            pltpu.VMEM((8, COLS), jnp.float32),
            pltpu.SemaphoreType.DMA,
            pltpu.SemaphoreType.DMA,
            pltpu.SemaphoreType.DMA,
            pltpu.SemaphoreType.DMA,
        ],
        compiler_params=pltpu.CompilerParams(needs_layout_passes=False,
                                             use_tc_tiling_on_sc=True),
    )
    return run(codes, table_flat, scales_flat)


def kernel(codes, codebooks, scales):
    table_flat = codebooks.reshape(CODEBOOK_NUM * CENTROIDS * 4)
    scales_flat = scales.reshape(NBLOCKS)
    return _dequant(codes, table_flat, scales_flat)
